# Initial kernel scaffold; baseline (speedup 1.0000x reference)
import jax
import jax.numpy as jnp
from jax.experimental import pallas as pl


def kernel(x, anchors0, sigma0, Wg0, anchors1, sigma1, Wg1, anchors2, sigma2, Wg2, anchors3, sigma3, Wg3):
    raise NotImplementedError("write your pallas kernel here")


# speedup vs baseline: 1.8032x; 1.8032x over previous
import jax
import jax.numpy as jnp
from jax.experimental import pallas as pl

_VS = (2, 4, 8, 32)
_VPAD = 128
_D = 512
_BN = 1000
_SEGS = tuple((32 * s, 32 * s + v) for s, v in enumerate(_VS))
_NEG = -1e30


def _assign_body(x_ref, inv2t_ref, winv2t_ref, t3_ref, q_ref, qtx_ref, qs_ref):
    x = x_ref[...]
    t1 = jnp.dot(x * x, inv2t_ref[...], preferred_element_type=jnp.float32)
    t2 = jnp.dot(x, winv2t_ref[...], preferred_element_type=jnp.float32)
    neg = -0.5 * (t1 - 2.0 * t2 + t3_ref[0:1, :])
    lane = jax.lax.broadcasted_iota(jnp.int32, neg.shape, 1)
    q = None
    for lo, hi in _SEGS:
        m = (lane >= lo) & (lane < hi)
        t = jnp.where(m, neg, _NEG)
        e = jnp.exp(t - jnp.max(t, axis=1, keepdims=True))
        e = jnp.where(m, e, 0.0)
        p = e / jnp.sum(e, axis=1, keepdims=True)
        q = p if q is None else q + p
    q_ref[...] = q

    @pl.when(pl.program_id(0) == 0)
    def _zero():
        qtx_ref[...] = jnp.zeros_like(qtx_ref)
        qs_ref[...] = jnp.zeros_like(qs_ref)

    qtx_ref[...] += jax.lax.dot_general(
        q, x, (((0,), (0,)), ((), ())), preferred_element_type=jnp.float32)
    qs_ref[...] += jax.lax.dot_general(
        q, jnp.ones((x.shape[0], 8), jnp.float32),
        (((0,), (0,)), ((), ())), preferred_element_type=jnp.float32)


def _vertex_body(qtx_ref, qs_ref, wc_ref, sig_ref, wg_ref, z2_ref):
    qsum = qs_ref[...][:, 0:1] + 1e-6
    z = (qtx_ref[...] / qsum - wc_ref[...]) / sig_ref[...]
    z2_ref[...] = jnp.zeros_like(z2_ref)
    for s, (lo, hi) in enumerate(_SEGS):
        zs = z[lo:hi, :]
        zs = zs / (jnp.sqrt(jnp.sum(zs * zs, axis=1, keepdims=True)) + 1e-6)
        g = jax.lax.dot_general(
            zs, zs, (((1,), (1,)), ((), ())), preferred_element_type=jnp.float32)
        g = g - jnp.max(g, axis=1, keepdims=True)
        a = jnp.exp(g)
        a = a / jnp.sum(a, axis=1, keepdims=True)
        az = jnp.dot(a, zs, preferred_element_type=jnp.float32)
        z2 = jnp.maximum(
            jnp.dot(az, wg_ref[s], preferred_element_type=jnp.float32), 0.0)
        z2_ref[lo:hi, _D * s:_D * (s + 1)] = z2


def _proj_body(x_ref, q_ref, z2_ref, o_ref):
    o_ref[:, 0:_D] = x_ref[...]
    o_ref[:, _D:] = jnp.dot(
        q_ref[...], z2_ref[...], preferred_element_type=jnp.float32)


def kernel(x, anchors0, sigma0, Wg0, anchors1, sigma1, Wg1,
           anchors2, sigma2, Wg2, anchors3, sigma3, Wg3):
    params = ((anchors0, sigma0), (anchors1, sigma1),
              (anchors2, sigma2), (anchors3, sigma3))
    n, d = x.shape
    nb = n // _BN

    inv2t = jnp.zeros((d, _VPAD), jnp.float32)
    winv2t = jnp.zeros((d, _VPAD), jnp.float32)
    t3 = jnp.zeros((_VPAD,), jnp.float32)
    wc = jnp.zeros((_VPAD, d), jnp.float32)
    sigc = jnp.ones((_VPAD, d), jnp.float32)
    for s, (w, sg) in enumerate(params):
        lo, hi = _SEGS[s]
        sig = jnp.abs(sg) + 1e-4
        inv2 = 1.0 / (sig * sig)
        inv2t = inv2t.at[:, lo:hi].set(inv2.T)
        winv2t = winv2t.at[:, lo:hi].set((w * inv2).T)
        t3 = t3.at[lo:hi].set(jnp.sum(w * w * inv2, axis=-1))
        wc = wc.at[lo:hi].set(w)
        sigc = sigc.at[lo:hi].set(sig)
    t3b = jnp.broadcast_to(t3[None, :], (8, _VPAD))
    wg = jnp.stack((Wg0, Wg1, Wg2, Wg3))

    q, qtx, qs = pl.pallas_call(
        _assign_body,
        grid=(nb,),
        in_specs=[
            pl.BlockSpec((_BN, d), lambda i: (i, 0)),
            pl.BlockSpec((d, _VPAD), lambda i: (0, 0)),
            pl.BlockSpec((d, _VPAD), lambda i: (0, 0)),
            pl.BlockSpec((8, _VPAD), lambda i: (0, 0)),
        ],
        out_specs=[
            pl.BlockSpec((_BN, _VPAD), lambda i: (i, 0)),
            pl.BlockSpec((_VPAD, d), lambda i: (0, 0)),
            pl.BlockSpec((_VPAD, 8), lambda i: (0, 0)),
        ],
        out_shape=[
            jax.ShapeDtypeStruct((n, _VPAD), jnp.float32),
            jax.ShapeDtypeStruct((_VPAD, d), jnp.float32),
            jax.ShapeDtypeStruct((_VPAD, 8), jnp.float32),
        ],
    )(x, inv2t, winv2t, t3b)

    z2 = pl.pallas_call(
        _vertex_body,
        out_shape=jax.ShapeDtypeStruct((_VPAD, 4 * d), jnp.float32),
    )(qtx, qs, wc, sigc, wg)

    out = pl.pallas_call(
        _proj_body,
        grid=(nb,),
        in_specs=[
            pl.BlockSpec((_BN, d), lambda i: (i, 0)),
            pl.BlockSpec((_BN, _VPAD), lambda i: (i, 0)),
            pl.BlockSpec((_VPAD, 4 * d), lambda i: (0, 0)),
        ],
        out_specs=pl.BlockSpec((_BN, 5 * d), lambda i: (i, 0)),
        out_shape=jax.ShapeDtypeStruct((n, 5 * d), jnp.float32),
    )(x, q, z2)
    return out


# speedup vs baseline: 1.9510x; 1.0820x over previous
import jax
import jax.numpy as jnp
from jax.experimental import pallas as pl

_VS = (2, 4, 8, 32)
_VPAD = 128
_D = 512
_BN = 1000
_SEGS = tuple((32 * s, 32 * s + v) for s, v in enumerate(_VS))
_NEG = -1e30


def _assign_body(x_ref, nhinv2t_ref, winv2t_ref, t3_ref, q_ref, qtx_ref, qs_ref):
    x = x_ref[...]
    t1 = jnp.dot(x * x, nhinv2t_ref[...], preferred_element_type=jnp.float32)
    t2 = jnp.dot(x, winv2t_ref[...], preferred_element_type=jnp.float32)
    neg = t1 + t2 + t3_ref[0:1, :]
    lane = jax.lax.broadcasted_iota(jnp.int32, neg.shape, 1)
    mval = jnp.full_like(neg, 1e30)
    for lo, hi in _SEGS:
        m = (lane >= lo) & (lane < hi)
        t = jnp.where(m, neg, _NEG)
        mx = jnp.max(t, axis=1, keepdims=True)
        mval = jnp.where(m, jnp.broadcast_to(mx, neg.shape), mval)
    e = jnp.exp(neg - mval)
    gk = jax.lax.broadcasted_iota(jnp.int32, (_VPAD, _VPAD), 0) // 32
    gl = jax.lax.broadcasted_iota(jnp.int32, (_VPAD, _VPAD), 1) // 32
    seg_ones = (gk == gl).astype(jnp.float32)
    esum = jnp.dot(e, seg_ones, preferred_element_type=jnp.float32)
    q = e / esum
    qb = q.astype(jnp.bfloat16)
    q_ref[...] = qb

    @pl.when(pl.program_id(0) == 0)
    def _zero():
        qtx_ref[...] = jnp.zeros_like(qtx_ref)
        qs_ref[...] = jnp.zeros_like(qs_ref)

    qtx_ref[...] += jax.lax.dot_general(
        qb, x.astype(jnp.bfloat16), (((0,), (0,)), ((), ())),
        preferred_element_type=jnp.float32)
    qs_ref[...] += jax.lax.dot_general(
        q, jnp.ones((x.shape[0], 8), jnp.float32),
        (((0,), (0,)), ((), ())), preferred_element_type=jnp.float32)


def _vertex_body(qtx_ref, qs_ref, wc_ref, sig_ref, wg_ref, z2_ref):
    qsum = qs_ref[...][:, 0:1] + 1e-6
    z = (qtx_ref[...] / qsum - wc_ref[...]) / sig_ref[...]
    z2_ref[...] = jnp.zeros_like(z2_ref)
    for s, (lo, hi) in enumerate(_SEGS):
        zs = z[lo:hi, :]
        zs = zs / (jnp.sqrt(jnp.sum(zs * zs, axis=1, keepdims=True)) + 1e-6)
        g = jax.lax.dot_general(
            zs, zs, (((1,), (1,)), ((), ())), preferred_element_type=jnp.float32)
        g = g - jnp.max(g, axis=1, keepdims=True)
        a = jnp.exp(g)
        a = a / jnp.sum(a, axis=1, keepdims=True)
        az = jnp.dot(a, zs, preferred_element_type=jnp.float32)
        z2 = jnp.maximum(
            jnp.dot(az, wg_ref[s], preferred_element_type=jnp.float32), 0.0)
        z2_ref[lo:hi, _D * s:_D * (s + 1)] = z2.astype(jnp.bfloat16)


def _proj_body(x_ref, q_ref, z2_ref, o_ref):
    o_ref[:, 0:_D] = x_ref[...]
    o_ref[:, _D:] = jnp.dot(
        q_ref[...], z2_ref[...], preferred_element_type=jnp.float32)


def kernel(x, anchors0, sigma0, Wg0, anchors1, sigma1, Wg1,
           anchors2, sigma2, Wg2, anchors3, sigma3, Wg3):
    params = ((anchors0, sigma0), (anchors1, sigma1),
              (anchors2, sigma2), (anchors3, sigma3))
    n, d = x.shape
    nb = n // _BN

    inv2t = jnp.zeros((d, _VPAD), jnp.float32)
    winv2t = jnp.zeros((d, _VPAD), jnp.float32)
    t3 = jnp.full((_VPAD,), _NEG, jnp.float32)
    wc = jnp.zeros((_VPAD, d), jnp.float32)
    sigc = jnp.ones((_VPAD, d), jnp.float32)
    for s, (w, sg) in enumerate(params):
        lo, hi = _SEGS[s]
        sig = jnp.abs(sg) + 1e-4
        inv2 = 1.0 / (sig * sig)
        inv2t = inv2t.at[:, lo:hi].set(-0.5 * inv2.T)
        winv2t = winv2t.at[:, lo:hi].set((w * inv2).T)
        t3 = t3.at[lo:hi].set(-0.5 * jnp.sum(w * w * inv2, axis=-1))
        wc = wc.at[lo:hi].set(w)
        sigc = sigc.at[lo:hi].set(sig)
    t3b = jnp.broadcast_to(t3[None, :], (8, _VPAD))
    wg = jnp.stack((Wg0, Wg1, Wg2, Wg3))

    q, qtx, qs = pl.pallas_call(
        _assign_body,
        grid=(nb,),
        in_specs=[
            pl.BlockSpec((_BN, d), lambda i: (i, 0)),
            pl.BlockSpec((d, _VPAD), lambda i: (0, 0)),
            pl.BlockSpec((d, _VPAD), lambda i: (0, 0)),
            pl.BlockSpec((8, _VPAD), lambda i: (0, 0)),
        ],
        out_specs=[
            pl.BlockSpec((_BN, _VPAD), lambda i: (i, 0)),
            pl.BlockSpec((_VPAD, d), lambda i: (0, 0)),
            pl.BlockSpec((_VPAD, 8), lambda i: (0, 0)),
        ],
        out_shape=[
            jax.ShapeDtypeStruct((n, _VPAD), jnp.bfloat16),
            jax.ShapeDtypeStruct((_VPAD, d), jnp.float32),
            jax.ShapeDtypeStruct((_VPAD, 8), jnp.float32),
        ],
    )(x, inv2t, winv2t, t3b)

    z2 = pl.pallas_call(
        _vertex_body,
        out_shape=jax.ShapeDtypeStruct((_VPAD, 4 * d), jnp.bfloat16),
    )(qtx, qs, wc, sigc, wg)

    out = pl.pallas_call(
        _proj_body,
        grid=(nb,),
        in_specs=[
            pl.BlockSpec((_BN, d), lambda i: (i, 0)),
            pl.BlockSpec((_BN, _VPAD), lambda i: (i, 0)),
            pl.BlockSpec((_VPAD, 4 * d), lambda i: (0, 0)),
        ],
        out_specs=pl.BlockSpec((_BN, 5 * d), lambda i: (i, 0)),
        out_shape=jax.ShapeDtypeStruct((n, 5 * d), jnp.float32),
    )(x, q, z2)
    return out


# speedup vs baseline: 2.1789x; 1.1168x over previous
import jax
import jax.numpy as jnp
from jax.experimental import pallas as pl
from jax.experimental.pallas import tpu as pltpu

_VS = (2, 4, 8, 32)
_VPAD = 128
_D = 512
_BN = 1000
_SEGS = tuple((32 * s, 32 * s + v) for s, v in enumerate(_VS))
_NEG = -1e30


def _gcu_body(x_ref, nhinv2t_ref, winv2t_ref, t3_ref, wc_ref, sig_ref, wg_ref,
              o_ref, xs_ref, q_ref, qtx_ref, qs_ref, z2_ref):
    i = pl.program_id(0)
    nb = pl.num_programs(0) // 2

    @pl.when(i == 0)
    def _zero():
        qtx_ref[...] = jnp.zeros_like(qtx_ref)
        qs_ref[...] = jnp.zeros_like(qs_ref)

    @pl.when(i < nb)
    def _assign():
        x = x_ref[...]
        t1 = jnp.dot(x * x, nhinv2t_ref[...], preferred_element_type=jnp.float32)
        t2 = jnp.dot(x, winv2t_ref[...], preferred_element_type=jnp.float32)
        neg = t1 + t2 + t3_ref[0:1, :]
        lane = jax.lax.broadcasted_iota(jnp.int32, neg.shape, 1)
        mval = jnp.full_like(neg, 1e30)
        for lo, hi in _SEGS:
            m = (lane >= lo) & (lane < hi)
            t = jnp.where(m, neg, _NEG)
            mx = jnp.max(t, axis=1, keepdims=True)
            mval = jnp.where(m, jnp.broadcast_to(mx, neg.shape), mval)
        e = jnp.exp(neg - mval)
        gk = jax.lax.broadcasted_iota(jnp.int32, (_VPAD, _VPAD), 0) // 32
        gl = jax.lax.broadcasted_iota(jnp.int32, (_VPAD, _VPAD), 1) // 32
        seg_ones = (gk == gl).astype(jnp.float32)
        esum = jnp.dot(e, seg_ones, preferred_element_type=jnp.float32)
        q = e / esum
        qb = q.astype(jnp.bfloat16)
        base = i * _BN
        xs_ref[pl.ds(base, _BN), :] = x
        q_ref[pl.ds(base, _BN), :] = qb
        qtx_ref[...] += jax.lax.dot_general(
            qb, x.astype(jnp.bfloat16), (((0,), (0,)), ((), ())),
            preferred_element_type=jnp.float32)
        qs_ref[...] += jax.lax.dot_general(
            q, jnp.ones((_BN, 8), jnp.float32),
            (((0,), (0,)), ((), ())), preferred_element_type=jnp.float32)

    @pl.when(i == nb)
    def _vertex():
        qsum = qs_ref[...][:, 0:1] + 1e-6
        z = (qtx_ref[...] / qsum - wc_ref[...]) / sig_ref[...]
        z2_ref[...] = jnp.zeros_like(z2_ref)
        for s, (lo, hi) in enumerate(_SEGS):
            zs = z[lo:hi, :]
            zs = zs / (jnp.sqrt(jnp.sum(zs * zs, axis=1, keepdims=True)) + 1e-6)
            g = jax.lax.dot_general(
                zs, zs, (((1,), (1,)), ((), ())),
                preferred_element_type=jnp.float32)
            g = g - jnp.max(g, axis=1, keepdims=True)
            a = jnp.exp(g)
            a = a / jnp.sum(a, axis=1, keepdims=True)
            az = jnp.dot(a, zs, preferred_element_type=jnp.float32)
            z2 = jnp.maximum(
                jnp.dot(az, wg_ref[s], preferred_element_type=jnp.float32), 0.0)
            z2_ref[lo:hi, _D * s:_D * (s + 1)] = z2.astype(jnp.bfloat16)

    @pl.when(i >= nb)
    def _project():
        base = (i - nb) * _BN
        o_ref[:, 0:_D] = xs_ref[pl.ds(base, _BN), :]
        o_ref[:, _D:] = jnp.dot(
            q_ref[pl.ds(base, _BN), :], z2_ref[...],
            preferred_element_type=jnp.float32)


def kernel(x, anchors0, sigma0, Wg0, anchors1, sigma1, Wg1,
           anchors2, sigma2, Wg2, anchors3, sigma3, Wg3):
    params = ((anchors0, sigma0), (anchors1, sigma1),
              (anchors2, sigma2), (anchors3, sigma3))
    n, d = x.shape
    nb = n // _BN

    inv2t = jnp.zeros((d, _VPAD), jnp.float32)
    winv2t = jnp.zeros((d, _VPAD), jnp.float32)
    t3 = jnp.full((_VPAD,), _NEG, jnp.float32)
    wc = jnp.zeros((_VPAD, d), jnp.float32)
    sigc = jnp.ones((_VPAD, d), jnp.float32)
    for s, (w, sg) in enumerate(params):
        lo, hi = _SEGS[s]
        sig = jnp.abs(sg) + 1e-4
        inv2 = 1.0 / (sig * sig)
        inv2t = inv2t.at[:, lo:hi].set(-0.5 * inv2.T)
        winv2t = winv2t.at[:, lo:hi].set((w * inv2).T)
        t3 = t3.at[lo:hi].set(-0.5 * jnp.sum(w * w * inv2, axis=-1))
        wc = wc.at[lo:hi].set(w)
        sigc = sigc.at[lo:hi].set(sig)
    t3b = jnp.broadcast_to(t3[None, :], (8, _VPAD))
    wg = jnp.stack((Wg0, Wg1, Wg2, Wg3))

    out = pl.pallas_call(
        _gcu_body,
        grid=(2 * nb,),
        in_specs=[
            pl.BlockSpec((_BN, d), lambda i: (jnp.minimum(i, nb - 1), 0)),
            pl.BlockSpec((d, _VPAD), lambda i: (0, 0)),
            pl.BlockSpec((d, _VPAD), lambda i: (0, 0)),
            pl.BlockSpec((8, _VPAD), lambda i: (0, 0)),
            pl.BlockSpec((_VPAD, d), lambda i: (0, 0)),
            pl.BlockSpec((_VPAD, d), lambda i: (0, 0)),
            pl.BlockSpec((4, d, d), lambda i: (0, 0, 0)),
        ],
        out_specs=pl.BlockSpec((_BN, 5 * d), lambda i: (jnp.maximum(i - nb, 0), 0)),
        out_shape=jax.ShapeDtypeStruct((n, 5 * d), jnp.float32),
        scratch_shapes=[
            pltpu.VMEM((n, d), jnp.float32),
            pltpu.VMEM((n, _VPAD), jnp.bfloat16),
            pltpu.VMEM((_VPAD, d), jnp.float32),
            pltpu.VMEM((_VPAD, 8), jnp.float32),
            pltpu.VMEM((_VPAD, 4 * d), jnp.bfloat16),
        ],
    )(x, inv2t, winv2t, t3b, wc, sigc, wg)
    return out


# speedup vs baseline: 2.3431x; 1.0754x over previous
import jax
import jax.numpy as jnp
from jax.experimental import pallas as pl
from jax.experimental.pallas import tpu as pltpu

_VS = (2, 4, 8, 32)
_VPAD = 128
_D = 512
_BN = 2000
_SEGS = tuple((32 * s, 32 * s + v) for s, v in enumerate(_VS))
_NEG = -1e30


def _gcu_body(x_ref, nhinv2t_ref, winv2t_ref, t3_ref, wc_ref, sig_ref, wg_ref,
              o_ref, q_ref, qtx_ref, qs_ref, z2_ref):
    i = pl.program_id(0)
    nb = pl.num_programs(0) // 2

    @pl.when(i == 0)
    def _zero():
        qtx_ref[...] = jnp.zeros_like(qtx_ref)
        qs_ref[...] = jnp.zeros_like(qs_ref)

    @pl.when(i < nb)
    def _assign():
        x = x_ref[...]
        t1 = jnp.dot(x * x, nhinv2t_ref[...], preferred_element_type=jnp.float32)
        t2 = jnp.dot(x, winv2t_ref[...], preferred_element_type=jnp.float32)
        neg = t1 + t2 + t3_ref[0:1, :]
        lane = jax.lax.broadcasted_iota(jnp.int32, neg.shape, 1)
        mval = jnp.full_like(neg, 1e30)
        for lo, hi in _SEGS:
            m = (lane >= lo) & (lane < hi)
            t = jnp.where(m, neg, _NEG)
            mx = jnp.max(t, axis=1, keepdims=True)
            mval = jnp.where(m, jnp.broadcast_to(mx, neg.shape), mval)
        e = jnp.exp(neg - mval)
        gk = jax.lax.broadcasted_iota(jnp.int32, (_VPAD, _VPAD), 0) // 32
        gl = jax.lax.broadcasted_iota(jnp.int32, (_VPAD, _VPAD), 1) // 32
        seg_ones = (gk == gl).astype(jnp.float32)
        esum = jnp.dot(e, seg_ones, preferred_element_type=jnp.float32)
        q = e / esum
        qb = q.astype(jnp.bfloat16)
        base = i * _BN
        q_ref[pl.ds(base, _BN), :] = q.astype(jnp.bfloat16)
        qtx_ref[...] += jax.lax.dot_general(
            qb, x.astype(jnp.bfloat16), (((0,), (0,)), ((), ())),
            preferred_element_type=jnp.float32)
        qs_ref[...] += jax.lax.dot_general(
            q, jnp.ones((_BN, 8), jnp.float32),
            (((0,), (0,)), ((), ())), preferred_element_type=jnp.float32)

    @pl.when(i == nb)
    def _vertex():
        qsum = qs_ref[...][:, 0:1] + 1e-6
        z = (qtx_ref[...] / qsum - wc_ref[...]) / sig_ref[...]
        z2_ref[...] = jnp.zeros_like(z2_ref)
        for s, (lo, hi) in enumerate(_SEGS):
            zs = z[lo:hi, :]
            zs = zs / (jnp.sqrt(jnp.sum(zs * zs, axis=1, keepdims=True)) + 1e-6)
            g = jax.lax.dot_general(
                zs, zs, (((1,), (1,)), ((), ())),
                preferred_element_type=jnp.float32)
            g = g - jnp.max(g, axis=1, keepdims=True)
            a = jnp.exp(g)
            a = a / jnp.sum(a, axis=1, keepdims=True)
            az = jnp.dot(a, zs, preferred_element_type=jnp.float32)
            z2 = jnp.maximum(
                jnp.dot(az.astype(jnp.bfloat16), wg_ref[s],
                        preferred_element_type=jnp.float32), 0.0)
            z2_ref[lo:hi, _D * s:_D * (s + 1)] = z2.astype(jnp.bfloat16)

    @pl.when(i >= nb)
    def _project():
        base = (i - nb) * _BN
        o_ref[:, 0:_D] = x_ref[...]
        o_ref[:, _D:] = jnp.dot(
            q_ref[pl.ds(base, _BN), :], z2_ref[...],
            preferred_element_type=jnp.float32)


def kernel(x, anchors0, sigma0, Wg0, anchors1, sigma1, Wg1,
           anchors2, sigma2, Wg2, anchors3, sigma3, Wg3):
    params = ((anchors0, sigma0), (anchors1, sigma1),
              (anchors2, sigma2), (anchors3, sigma3))
    n, d = x.shape
    nb = n // _BN

    inv2t = jnp.zeros((d, _VPAD), jnp.float32)
    winv2t = jnp.zeros((d, _VPAD), jnp.float32)
    t3 = jnp.full((_VPAD,), _NEG, jnp.float32)
    wc = jnp.zeros((_VPAD, d), jnp.float32)
    sigc = jnp.ones((_VPAD, d), jnp.float32)
    for s, (w, sg) in enumerate(params):
        lo, hi = _SEGS[s]
        sig = jnp.abs(sg) + 1e-4
        inv2 = 1.0 / (sig * sig)
        inv2t = inv2t.at[:, lo:hi].set(-0.5 * inv2.T)
        winv2t = winv2t.at[:, lo:hi].set((w * inv2).T)
        t3 = t3.at[lo:hi].set(-0.5 * jnp.sum(w * w * inv2, axis=-1))
        wc = wc.at[lo:hi].set(w)
        sigc = sigc.at[lo:hi].set(sig)
    t3b = jnp.broadcast_to(t3[None, :], (8, _VPAD))
    wg = jnp.stack((Wg0, Wg1, Wg2, Wg3)).astype(jnp.bfloat16)

    out = pl.pallas_call(
        _gcu_body,
        grid=(2 * nb,),
        in_specs=[
            pl.BlockSpec((_BN, d), lambda i: (jnp.where(i < nb, i, i - nb), 0)),
            pl.BlockSpec((d, _VPAD), lambda i: (0, 0)),
            pl.BlockSpec((d, _VPAD), lambda i: (0, 0)),
            pl.BlockSpec((8, _VPAD), lambda i: (0, 0)),
            pl.BlockSpec((_VPAD, d), lambda i: (0, 0)),
            pl.BlockSpec((_VPAD, d), lambda i: (0, 0)),
            pl.BlockSpec((4, d, d), lambda i: (0, 0, 0)),
        ],
        out_specs=pl.BlockSpec((_BN, 5 * d), lambda i: (jnp.maximum(i - nb, 0), 0)),
        out_shape=jax.ShapeDtypeStruct((n, 5 * d), jnp.float32),
        scratch_shapes=[
            pltpu.VMEM((n, _VPAD), jnp.bfloat16),
            pltpu.VMEM((_VPAD, d), jnp.float32),
            pltpu.VMEM((_VPAD, 8), jnp.float32),
            pltpu.VMEM((_VPAD, 4 * d), jnp.bfloat16),
        ],
    )(x, inv2t, winv2t, t3b, wc, sigc, wg)
    return out


# speedup vs baseline: 2.8681x; 1.2241x over previous
import jax
import jax.numpy as jnp
from jax.experimental import pallas as pl
from jax.experimental.pallas import tpu as pltpu

_VS = (2, 4, 8, 32)
_VPAD = 128
_D = 512
_BN = 2000
_SEGS = tuple((32 * s, 32 * s + v) for s, v in enumerate(_VS))
_NEG = -1e30
_DIMS_RR = (((1,), (1,)), ((), ()))
_DIMS_CC = (((0,), (0,)), ((), ()))


def _gcu_body(x_ref, a0_ref, s0_ref, g0_ref, a1_ref, s1_ref, g1_ref,
              a2_ref, s2_ref, g2_ref, a3_ref, s3_ref, g3_ref,
              o_ref, q_ref, qtx_ref, qs_ref, z2_ref,
              wpk_ref, spk_ref, nh_ref, wi_ref, t3_ref):
    i = pl.program_id(0)
    nb = pl.num_programs(0) // 2

    @pl.when(i == 0)
    def _prep():
        spk_ref[...] = jnp.ones_like(spk_ref)
        wpk_ref[...] = jnp.zeros_like(wpk_ref)
        for (lo, hi), a_ref, s_ref in ((_SEGS[0], a0_ref, s0_ref),
                                       (_SEGS[1], a1_ref, s1_ref),
                                       (_SEGS[2], a2_ref, s2_ref),
                                       (_SEGS[3], a3_ref, s3_ref)):
            wpk_ref[lo:hi, :] = a_ref[...]
            spk_ref[lo:hi, :] = jnp.abs(s_ref[...]) + 1e-4
        sig = spk_ref[...]
        w = wpk_ref[...]
        inv2 = 1.0 / (sig * sig)
        nh_ref[...] = -0.5 * inv2
        wi_ref[...] = w * inv2
        t3 = jax.lax.dot_general(
            jnp.ones((8, _D), jnp.float32), -0.5 * (w * w) * inv2, _DIMS_RR,
            preferred_element_type=jnp.float32)
        lane = jax.lax.broadcasted_iota(jnp.int32, (8, _VPAD), 1)
        within = lane % 32
        group = lane // 32
        vlim = jnp.where(group == 0, _VS[0],
                         jnp.where(group == 1, _VS[1],
                                   jnp.where(group == 2, _VS[2], _VS[3])))
        t3_ref[...] = jnp.where(within >= vlim, _NEG, t3)
        qtx_ref[...] = jnp.zeros_like(qtx_ref)
        qs_ref[...] = jnp.zeros_like(qs_ref)

    @pl.when(i < nb)
    def _assign():
        x = x_ref[...]
        t1 = jax.lax.dot_general(x * x, nh_ref[...], _DIMS_RR,
                                 preferred_element_type=jnp.float32)
        t2 = jax.lax.dot_general(x, wi_ref[...], _DIMS_RR,
                                 preferred_element_type=jnp.float32)
        neg = t1 + t2 + t3_ref[0:1, :]
        lane = jax.lax.broadcasted_iota(jnp.int32, neg.shape, 1)
        mval = jnp.full_like(neg, 1e30)
        for lo, hi in _SEGS:
            m = (lane >= lo) & (lane < hi)
            t = jnp.where(m, neg, _NEG)
            mx = jnp.max(t, axis=1, keepdims=True)
            mval = jnp.where(m, jnp.broadcast_to(mx, neg.shape), mval)
        e = jnp.exp(neg - mval)
        gk = jax.lax.broadcasted_iota(jnp.int32, (_VPAD, _VPAD), 0) // 32
        gl = jax.lax.broadcasted_iota(jnp.int32, (_VPAD, _VPAD), 1) // 32
        seg_ones = (gk == gl).astype(jnp.float32)
        esum = jnp.dot(e, seg_ones, preferred_element_type=jnp.float32)
        q = e / esum
        qb = q.astype(jnp.bfloat16)
        q_ref[pl.ds(i * _BN, _BN), :] = qb
        qtx_ref[...] += jax.lax.dot_general(
            qb, x.astype(jnp.bfloat16), _DIMS_CC,
            preferred_element_type=jnp.float32)
        qs_ref[...] += jax.lax.dot_general(
            q, jnp.ones((_BN, 8), jnp.float32), _DIMS_CC,
            preferred_element_type=jnp.float32)

    @pl.when(i == nb)
    def _vertex():
        qsum = qs_ref[...][:, 0:1] + 1e-6
        z = (qtx_ref[...] / qsum - wpk_ref[...]) / spk_ref[...]
        z2_ref[...] = jnp.zeros_like(z2_ref)
        for s, (lo, hi) in enumerate(_SEGS):
            zs = z[lo:hi, :]
            zs = zs / (jnp.sqrt(jnp.sum(zs * zs, axis=1, keepdims=True)) + 1e-6)
            g = jax.lax.dot_general(zs, zs, _DIMS_RR,
                                    preferred_element_type=jnp.float32)
            g = g - jnp.max(g, axis=1, keepdims=True)
            a = jnp.exp(g)
            a = a / jnp.sum(a, axis=1, keepdims=True)
            az = jnp.dot(a, zs, preferred_element_type=jnp.float32)
            wg = (g0_ref, g1_ref, g2_ref, g3_ref)[s][...].astype(jnp.bfloat16)
            z2 = jnp.maximum(
                jnp.dot(az.astype(jnp.bfloat16), wg,
                        preferred_element_type=jnp.float32), 0.0)
            z2_ref[lo:hi, _D * s:_D * (s + 1)] = z2.astype(jnp.bfloat16)

    @pl.when(i >= nb)
    def _project():
        o_ref[:, 0:_D] = x_ref[...]
        o_ref[:, _D:] = jnp.dot(
            q_ref[pl.ds((i - nb) * _BN, _BN), :], z2_ref[...],
            preferred_element_type=jnp.float32)


def kernel(x, anchors0, sigma0, Wg0, anchors1, sigma1, Wg1,
           anchors2, sigma2, Wg2, anchors3, sigma3, Wg3):
    n, d = x.shape
    nb = n // _BN

    def _full(shape):
        nd = len(shape)
        return pl.BlockSpec(shape, lambda i, _nd=nd: (0,) * _nd)

    out = pl.pallas_call(
        _gcu_body,
        grid=(2 * nb,),
        in_specs=[
            pl.BlockSpec((_BN, d), lambda i: (jnp.where(i < nb, i, i - nb), 0)),
            _full(anchors0.shape), _full(sigma0.shape), _full(Wg0.shape),
            _full(anchors1.shape), _full(sigma1.shape), _full(Wg1.shape),
            _full(anchors2.shape), _full(sigma2.shape), _full(Wg2.shape),
            _full(anchors3.shape), _full(sigma3.shape), _full(Wg3.shape),
        ],
        out_specs=pl.BlockSpec((_BN, 5 * d),
                               lambda i: (jnp.maximum(i - nb, 0), 0)),
        out_shape=jax.ShapeDtypeStruct((n, 5 * d), jnp.float32),
        scratch_shapes=[
            pltpu.VMEM((n, _VPAD), jnp.bfloat16),
            pltpu.VMEM((_VPAD, d), jnp.float32),
            pltpu.VMEM((_VPAD, 8), jnp.float32),
            pltpu.VMEM((_VPAD, 4 * d), jnp.bfloat16),
            pltpu.VMEM((_VPAD, d), jnp.float32),
            pltpu.VMEM((_VPAD, d), jnp.float32),
            pltpu.VMEM((_VPAD, d), jnp.float32),
            pltpu.VMEM((_VPAD, d), jnp.float32),
            pltpu.VMEM((8, _VPAD), jnp.float32),
        ],
    )(x, anchors0, sigma0, Wg0, anchors1, sigma1, Wg1,
      anchors2, sigma2, Wg2, anchors3, sigma3, Wg3)
    return out


# speedup vs baseline: 2.9139x; 1.0160x over previous
import jax
import jax.numpy as jnp
from jax.experimental import pallas as pl
from jax.experimental.pallas import tpu as pltpu

_VS = (2, 4, 8, 32)
_VPAD = 128
_D = 512
_BN = 1000
_SEGS = tuple((32 * s, 32 * s + v) for s, v in enumerate(_VS))
_NEG = -1e30
_DIMS_RR = (((1,), (1,)), ((), ()))
_DIMS_CC = (((0,), (0,)), ((), ()))


def _gcu_body(x_ref, a0_ref, s0_ref, g0_ref, a1_ref, s1_ref, g1_ref,
              a2_ref, s2_ref, g2_ref, a3_ref, s3_ref, g3_ref,
              o_ref, q_ref, xs_ref, qtx_ref, qs_ref, z2_ref,
              wpk_ref, spk_ref, nh_ref, wi_ref, t3_ref):
    i = pl.program_id(0)
    nb = pl.num_programs(0) // 2

    @pl.when(i == 0)
    def _prep():
        spk_ref[...] = jnp.ones_like(spk_ref)
        wpk_ref[...] = jnp.zeros_like(wpk_ref)
        for (lo, hi), a_ref, s_ref in ((_SEGS[0], a0_ref, s0_ref),
                                       (_SEGS[1], a1_ref, s1_ref),
                                       (_SEGS[2], a2_ref, s2_ref),
                                       (_SEGS[3], a3_ref, s3_ref)):
            wpk_ref[lo:hi, :] = a_ref[...]
            spk_ref[lo:hi, :] = jnp.abs(s_ref[...]) + 1e-4
        sig = spk_ref[...]
        w = wpk_ref[...]
        inv2 = 1.0 / (sig * sig)
        nh_ref[...] = -0.5 * inv2
        wi_ref[...] = w * inv2
        t3 = jax.lax.dot_general(
            jnp.ones((8, _D), jnp.float32), -0.5 * (w * w) * inv2, _DIMS_RR,
            preferred_element_type=jnp.float32)
        lane = jax.lax.broadcasted_iota(jnp.int32, (8, _VPAD), 1)
        within = lane % 32
        group = lane // 32
        vlim = jnp.where(group == 0, _VS[0],
                         jnp.where(group == 1, _VS[1],
                                   jnp.where(group == 2, _VS[2], _VS[3])))
        t3_ref[...] = jnp.where(within >= vlim, _NEG, t3)
        qtx_ref[...] = jnp.zeros_like(qtx_ref)
        qs_ref[...] = jnp.zeros_like(qs_ref)

    @pl.when(i < nb)
    def _assign():
        x = x_ref[...]
        t1 = jax.lax.dot_general(x * x, nh_ref[...], _DIMS_RR,
                                 preferred_element_type=jnp.float32)
        t2 = jax.lax.dot_general(x, wi_ref[...], _DIMS_RR,
                                 preferred_element_type=jnp.float32)
        neg = t1 + t2 + t3_ref[0:1, :]
        lane = jax.lax.broadcasted_iota(jnp.int32, neg.shape, 1)
        mval = jnp.full_like(neg, 1e30)
        for lo, hi in _SEGS:
            m = (lane >= lo) & (lane < hi)
            t = jnp.where(m, neg, _NEG)
            mx = jnp.max(t, axis=1, keepdims=True)
            mval = jnp.where(m, jnp.broadcast_to(mx, neg.shape), mval)
        e = jnp.exp(neg - mval)
        gk = jax.lax.broadcasted_iota(jnp.int32, (_VPAD, _VPAD), 0) // 32
        gl = jax.lax.broadcasted_iota(jnp.int32, (_VPAD, _VPAD), 1) // 32
        seg_ones = (gk == gl).astype(jnp.float32)
        esum = jnp.dot(e, seg_ones, preferred_element_type=jnp.float32)
        q = e / esum
        qb = q.astype(jnp.bfloat16)
        q_ref[pl.ds(i * _BN, _BN), :] = q
        xs_ref[pl.ds(i * _BN, _BN), :] = x.astype(jnp.bfloat16)
        qtx_ref[...] += jax.lax.dot_general(
            qb, x.astype(jnp.bfloat16), _DIMS_CC,
            preferred_element_type=jnp.float32)
        qs_ref[...] += jax.lax.dot_general(
            q, jnp.ones((_BN, 8), jnp.float32), _DIMS_CC,
            preferred_element_type=jnp.float32)

    @pl.when(i == nb)
    def _vertex():
        qsum = qs_ref[...][:, 0:1] + 1e-6
        z = (qtx_ref[...] / qsum - wpk_ref[...]) / spk_ref[...]
        z2_ref[...] = jnp.zeros_like(z2_ref)
        for s, (lo, hi) in enumerate(_SEGS):
            zs = z[lo:hi, :]
            zs = zs / (jnp.sqrt(jnp.sum(zs * zs, axis=1, keepdims=True)) + 1e-6)
            g = jax.lax.dot_general(zs, zs, _DIMS_RR,
                                    preferred_element_type=jnp.float32)
            g = g - jnp.max(g, axis=1, keepdims=True)
            a = jnp.exp(g)
            a = a / jnp.sum(a, axis=1, keepdims=True)
            az = jnp.dot(a, zs, preferred_element_type=jnp.float32)
            wg = (g0_ref, g1_ref, g2_ref, g3_ref)[s][...].astype(jnp.bfloat16)
            z2 = jnp.maximum(
                jnp.dot(az.astype(jnp.bfloat16), wg,
                        preferred_element_type=jnp.float32), 0.0)
            z2_ref[lo:hi, _D * s:_D * (s + 1)] = z2.astype(jnp.bfloat16)

    @pl.when(i >= nb)
    def _project():
        base = (i - nb) * _BN
        o_ref[:, 0:_D] = xs_ref[pl.ds(base, _BN), :].astype(jnp.float32)
        o_ref[:, _D:] = jnp.dot(
            q_ref[pl.ds(base, _BN), :].astype(jnp.bfloat16), z2_ref[...],
            preferred_element_type=jnp.float32)


def kernel(x, anchors0, sigma0, Wg0, anchors1, sigma1, Wg1,
           anchors2, sigma2, Wg2, anchors3, sigma3, Wg3):
    n, d = x.shape
    nb = n // _BN

    def _full(shape):
        nd = len(shape)
        return pl.BlockSpec(shape, lambda i, _nd=nd: (0,) * _nd)

    out = pl.pallas_call(
        _gcu_body,
        grid=(2 * nb,),
        in_specs=[
            pl.BlockSpec((_BN, d), lambda i: (jnp.minimum(i, nb - 1), 0)),
            _full(anchors0.shape), _full(sigma0.shape), _full(Wg0.shape),
            _full(anchors1.shape), _full(sigma1.shape), _full(Wg1.shape),
            _full(anchors2.shape), _full(sigma2.shape), _full(Wg2.shape),
            _full(anchors3.shape), _full(sigma3.shape), _full(Wg3.shape),
        ],
        out_specs=pl.BlockSpec((_BN, 5 * d),
                               lambda i: (jnp.maximum(i - nb, 0), 0)),
        out_shape=jax.ShapeDtypeStruct((n, 5 * d), jnp.float32),
        scratch_shapes=[
            pltpu.VMEM((n, _VPAD), jnp.float32),
            pltpu.VMEM((n, _D), jnp.bfloat16),
            pltpu.VMEM((_VPAD, d), jnp.float32),
            pltpu.VMEM((_VPAD, 8), jnp.float32),
            pltpu.VMEM((_VPAD, 4 * d), jnp.bfloat16),
            pltpu.VMEM((_VPAD, d), jnp.float32),
            pltpu.VMEM((_VPAD, d), jnp.float32),
            pltpu.VMEM((_VPAD, d), jnp.float32),
            pltpu.VMEM((_VPAD, d), jnp.float32),
            pltpu.VMEM((8, _VPAD), jnp.float32),
        ],
    )(x, anchors0, sigma0, Wg0, anchors1, sigma1, Wg1,
      anchors2, sigma2, Wg2, anchors3, sigma3, Wg3)
    return out
